# baseline (speedup 1.0000x reference)
import functools

import jax
import jax.numpy as jnp
from jax import lax
from jax.experimental import pallas as pl
from jax.experimental.pallas import tpu as pltpu
from jax.experimental.pallas import tpu_sc as plsc

BLK = 512
MP = 2560
EW = 128


def _rank_cell(s_blk_ref, s_all_ref, lab_blk_ref, rank_ref, ex_ref):
    bi = pl.program_id(0)
    si = s_blk_ref[0, 0, :][:, None]
    sall = s_all_ref[...]
    gt = (sall > si).astype(jnp.float32)
    jglob = jax.lax.broadcasted_iota(jnp.int32, gt.shape, 1)
    iglob = bi * BLK + jax.lax.broadcasted_iota(jnp.int32, gt.shape, 0)
    eq = ((sall == si) & (jglob < iglob)).astype(jnp.float32)
    rank_ref[0, 0, :] = jnp.sum(gt + eq, axis=1).astype(jnp.int32)

    lane = jax.lax.broadcasted_iota(jnp.int32, (BLK, EW), 1)
    row_i = (bi * BLK
             + jax.lax.broadcasted_iota(jnp.int32, (BLK, EW), 0)
             ).astype(jnp.float32)
    labc = lab_blk_ref[0, 0, :][:, None]
    ex = jnp.where(lane == 0, si, 0.0)
    ex = jnp.where(lane == 1, labc, ex)
    ex_ref[...] = jnp.where(lane == 2, row_i, ex)


def _ranks(scop, labp, npad, nb):
    return pl.pallas_call(
        _rank_cell,
        grid=(nb,),
        in_specs=[
            pl.BlockSpec((1, 1, BLK), lambda i: (i, 0, 0)),
            pl.BlockSpec((1, npad), lambda i: (0, 0)),
            pl.BlockSpec((1, 1, BLK), lambda i: (i, 0, 0)),
        ],
        out_specs=[
            pl.BlockSpec((1, 1, BLK), lambda i: (i, 0, 0)),
            pl.BlockSpec((BLK, EW), lambda i: (i, 0)),
        ],
        out_shape=[
            jax.ShapeDtypeStruct((nb, 1, BLK), jnp.int32),
            jax.ShapeDtypeStruct((npad, EW), jnp.float32),
        ],
        compiler_params=pltpu.CompilerParams(
            dimension_semantics=("arbitrary",)),
    )(scop.reshape(nb, 1, BLK), scop.reshape(1, npad),
      labp.reshape(nb, 1, BLK))


def _unpack_cell(w_ref, o_ref):
    w = w_ref[...]
    planes = [((w >> (8 * k)) & 1).astype(jnp.int8) for k in range(4)]
    o_ref[...] = jnp.concatenate(planes, axis=1)


def _unpack(smw, npad, nb):
    return pl.pallas_call(
        _unpack_cell,
        grid=(nb,),
        in_specs=[pl.BlockSpec((BLK, MP // 4), lambda i: (i, 0))],
        out_specs=pl.BlockSpec((BLK, MP), lambda i: (i, 0)),
        out_shape=jax.ShapeDtypeStruct((npad, MP), jnp.int8),
        compiler_params=pltpu.CompilerParams(
            dimension_semantics=("arbitrary",)),
    )(smw)


def _sc_sort_scatter(mask_words, extras, rank, npad):
    info = plsc.get_sparse_core_info()
    nw = info.num_cores * info.num_subcores
    rows_per_w = npad // nw
    chunk = 80
    nchunk = rows_per_w // chunk
    mw = mask_words.shape[1]
    ew = extras.shape[1]
    mesh = plsc.VectorSubcoreMesh(core_axis_name="c", subcore_axis_name="s")

    @functools.partial(
        pl.kernel, mesh=mesh,
        out_type=[
            jax.ShapeDtypeStruct((npad, mw), jnp.int32),
            jax.ShapeDtypeStruct((npad, ew), jnp.float32),
        ],
        scratch_types=[
            pltpu.VMEM((chunk,), jnp.int32),
            pltpu.VMEM((chunk, mw), jnp.int32),
            pltpu.VMEM((chunk, ew), jnp.float32),
            pltpu.SemaphoreType.DMA,
        ],
    )
    def scatter_k(mask_hbm, ex_hbm, rank_hbm, omask_hbm, oex_hbm,
                  idx_v, mrows_v, erows_v, sem):
        wid = lax.axis_index("s") * info.num_cores + lax.axis_index("c")
        for c in range(nchunk):
            base = wid * rows_per_w + c * chunk
            pltpu.sync_copy(rank_hbm.at[pl.ds(base, chunk)], idx_v)
            pltpu.sync_copy(mask_hbm.at[pl.ds(base, chunk)], mrows_v)
            pltpu.sync_copy(ex_hbm.at[pl.ds(base, chunk)], erows_v)
            cp1 = pltpu.async_copy(mrows_v, omask_hbm.at[idx_v], sem)
            cp2 = pltpu.async_copy(erows_v, oex_hbm.at[idx_v], sem)
            cp1.wait()
            cp2.wait()

    return scatter_k(mask_words, extras, rank)


def _nms_cell(labA_ref, labB_ref, scoB_ref, a_ref, b_ref,
              keep_out, ks_out, keep_scr, sup_ref, s_ref, kl_ref):
    bi = pl.program_id(0)
    bj = pl.program_id(1)

    @pl.when(bj == 0)
    def _init():
        sup_ref[...] = jnp.zeros_like(sup_ref)

    @pl.when(bj <= bi)
    def _work():
        A = a_ref[...].astype(jnp.bfloat16)
        B = b_ref[...].astype(jnp.bfloat16)
        inter = jax.lax.dot_general(
            A, B, dimension_numbers=(((1,), (1,)), ((), ())),
            preferred_element_type=jnp.float32)
        areaA = jnp.sum(A.astype(jnp.float32), axis=1)
        areaB = jnp.sum(B.astype(jnp.float32), axis=1)
        labA = labA_ref[0, 0, :]
        labB = labB_ref[0, 0, :]
        cond = (3.0 * inter > areaA[:, None] + areaB[None, :]) \
            & (labA[:, None] == labB[None, :])
        condf = cond.astype(jnp.float32)

        @pl.when(bj < bi)
        def _offdiag():
            kb = keep_scr[pl.ds(bj, 1), :]
            contrib = jax.lax.dot_general(
                kb, condf, dimension_numbers=(((1,), (0,)), ((), ())),
                preferred_element_type=jnp.float32)
            sup_ref[...] = sup_ref[...] + contrib

        @pl.when(bj == bi)
        def _diag():
            ii = jax.lax.broadcasted_iota(jnp.int32, (BLK, BLK), 0)
            jj = jax.lax.broadcasted_iota(jnp.int32, (BLK, BLK), 1)
            Sf = condf * (jj > ii).astype(jnp.float32)
            s_ref[...] = Sf
            kl_ref[...] = (sup_ref[...] == 0.0).astype(jnp.float32)

            @pl.when(jnp.max(Sf) > 0.0)
            def _serial():
                lane = jax.lax.broadcasted_iota(jnp.int32, (1, BLK), 1)

                def body(i, carry):
                    kl = kl_ref[...]
                    ki = jnp.sum(jnp.where(lane == i, kl, 0.0))
                    srow = s_ref[pl.ds(i, 1), :]
                    kl_ref[...] = kl * (1.0 - srow * ki)
                    return carry

                jax.lax.fori_loop(0, BLK, body, 0)

            kl = kl_ref[...]
            keep_scr[pl.ds(bi, 1), :] = kl
            keep_out[0, ...] = kl
            ks_out[0, ...] = kl * scoB_ref[0, ...]


def _run_nms(msp, labp, scop, nb):
    grid = (nb, nb)
    out_shape = [
        jax.ShapeDtypeStruct((nb, 1, BLK), jnp.float32),
        jax.ShapeDtypeStruct((nb, 1, BLK), jnp.float32),
    ]
    keep_f, ks = pl.pallas_call(
        _nms_cell,
        grid=grid,
        in_specs=[
            pl.BlockSpec((1, 1, BLK), lambda i, j: (jnp.minimum(i, j), 0, 0)),
            pl.BlockSpec((1, 1, BLK), lambda i, j: (i, 0, 0)),
            pl.BlockSpec((1, 1, BLK), lambda i, j: (i, 0, 0)),
            pl.BlockSpec((BLK, MP), lambda i, j: (jnp.minimum(i, j), 0)),
            pl.BlockSpec((BLK, MP), lambda i, j: (i, 0)),
        ],
        out_specs=[
            pl.BlockSpec((1, 1, BLK), lambda i, j: (i, 0, 0)),
            pl.BlockSpec((1, 1, BLK), lambda i, j: (i, 0, 0)),
        ],
        out_shape=out_shape,
        scratch_shapes=[
            pltpu.VMEM((nb, BLK), jnp.float32),
            pltpu.VMEM((1, BLK), jnp.float32),
            pltpu.VMEM((BLK, BLK), jnp.float32),
            pltpu.VMEM((1, BLK), jnp.float32),
        ],
        compiler_params=pltpu.CompilerParams(
            dimension_semantics=("arbitrary", "arbitrary")),
    )(labp, labp, scop, msp, msp)
    return keep_f, ks


def kernel(bev_masks, scores, labels):
    n, m = bev_masks.shape
    nb = (n + BLK - 1) // BLK
    npad = nb * BLK

    mwords = jnp.zeros((npad, MP // 4), jnp.int32)
    mwords = mwords.at[:n, :m // 4].set(jax.lax.bitcast_convert_type(
        bev_masks.astype(jnp.int8).reshape(n, m // 4, 4), jnp.int32))
    scop = jnp.full((npad,), -1.0, jnp.float32).at[:n].set(scores)
    labp = jnp.full((npad,), -1.0, jnp.float32).at[:n].set(
        labels.astype(jnp.float32))

    rank, extras = _ranks(scop, labp, npad, nb)

    smw, sex = _sc_sort_scatter(mwords, extras, rank.reshape(npad), npad)

    msp = _unpack(smw, npad, nb)

    scos = sex[:, 0]
    labs = sex[:, 1].astype(jnp.int32)
    order = sex[:n, 2].astype(jnp.int32)

    keep_f, ks = _run_nms(msp, labs.reshape(nb, 1, BLK),
                          scos.reshape(nb, 1, BLK), nb)

    keep = keep_f.reshape(npad)[:n] > 0.5
    kept_scores = ks.reshape(npad)[:n]
    return order, keep, kept_scores


# speedup vs baseline: 2.8471x; 2.8471x over previous
import functools

import jax
import jax.numpy as jnp
from jax import lax
from jax.experimental import pallas as pl
from jax.experimental.pallas import tpu as pltpu
from jax.experimental.pallas import tpu_sc as plsc

BLK = 512
MP = 2560
EW = 128


def _rank_cell(s_blk_ref, s_all_ref, lab_blk_ref, rank_ref, ex_ref):
    bi = pl.program_id(0)
    si = s_blk_ref[0, 0, :][:, None]
    sall = s_all_ref[...]
    gt = (sall > si).astype(jnp.float32)
    jglob = jax.lax.broadcasted_iota(jnp.int32, gt.shape, 1)
    iglob = bi * BLK + jax.lax.broadcasted_iota(jnp.int32, gt.shape, 0)
    eq = ((sall == si) & (jglob < iglob)).astype(jnp.float32)
    rank_ref[0, 0, :] = jnp.sum(gt + eq, axis=1).astype(jnp.int32)

    lane = jax.lax.broadcasted_iota(jnp.int32, (BLK, EW), 1)
    row_i = (bi * BLK
             + jax.lax.broadcasted_iota(jnp.int32, (BLK, EW), 0)
             ).astype(jnp.float32)
    labc = lab_blk_ref[0, 0, :][:, None]
    ex = jnp.where(lane == 0, si, 0.0)
    ex = jnp.where(lane == 1, labc, ex)
    ex_ref[...] = jnp.where(lane == 2, row_i, ex)


def _ranks(scop, labp, npad, nb):
    return pl.pallas_call(
        _rank_cell,
        grid=(nb,),
        in_specs=[
            pl.BlockSpec((1, 1, BLK), lambda i: (i, 0, 0)),
            pl.BlockSpec((1, npad), lambda i: (0, 0)),
            pl.BlockSpec((1, 1, BLK), lambda i: (i, 0, 0)),
        ],
        out_specs=[
            pl.BlockSpec((1, 1, BLK), lambda i: (i, 0, 0)),
            pl.BlockSpec((BLK, EW), lambda i: (i, 0)),
        ],
        out_shape=[
            jax.ShapeDtypeStruct((nb, 1, BLK), jnp.int32),
            jax.ShapeDtypeStruct((npad, EW), jnp.float32),
        ],
        compiler_params=pltpu.CompilerParams(
            dimension_semantics=("arbitrary",)),
    )(scop.reshape(nb, 1, BLK), scop.reshape(1, npad),
      labp.reshape(nb, 1, BLK))


def _pack(masks, n, m, npad, nb):
    qw = MP // 4

    def cell(m_ref, w_ref):
        bi = pl.program_id(0)
        x = m_ref[...].astype(jnp.int32)
        rglob = bi * BLK + jax.lax.broadcasted_iota(jnp.int32, (BLK, MP), 0)
        cglob = jax.lax.broadcasted_iota(jnp.int32, (BLK, MP), 1)
        x = jnp.where((rglob < n) & (cglob < m), x, 0)
        w_ref[...] = (x[:, 0:qw]
                      | (x[:, qw:2 * qw] << 8)
                      | (x[:, 2 * qw:3 * qw] << 16)
                      | (x[:, 3 * qw:4 * qw] << 24))

    return pl.pallas_call(
        cell,
        grid=(nb,),
        in_specs=[pl.BlockSpec((BLK, MP), lambda i: (i, 0))],
        out_specs=pl.BlockSpec((BLK, qw), lambda i: (i, 0)),
        out_shape=jax.ShapeDtypeStruct((npad, qw), jnp.int32),
        compiler_params=pltpu.CompilerParams(
            dimension_semantics=("arbitrary",)),
    )(masks)


def _unpack_cell(w_ref, o_ref):
    w = w_ref[...]
    planes = [((w >> (8 * k)) & 1).astype(jnp.int8) for k in range(4)]
    o_ref[...] = jnp.concatenate(planes, axis=1)


def _unpack(smw, npad, nb):
    return pl.pallas_call(
        _unpack_cell,
        grid=(nb,),
        in_specs=[pl.BlockSpec((BLK, MP // 4), lambda i: (i, 0))],
        out_specs=pl.BlockSpec((BLK, MP), lambda i: (i, 0)),
        out_shape=jax.ShapeDtypeStruct((npad, MP), jnp.int8),
        compiler_params=pltpu.CompilerParams(
            dimension_semantics=("arbitrary",)),
    )(smw)


def _sc_sort_scatter(mask_words, extras, rank, npad):
    info = plsc.get_sparse_core_info()
    nw = info.num_cores * info.num_subcores
    rows_per_w = npad // nw
    chunk = 80
    nchunk = rows_per_w // chunk
    mw = mask_words.shape[1]
    ew = extras.shape[1]
    mesh = plsc.VectorSubcoreMesh(core_axis_name="c", subcore_axis_name="s")

    @functools.partial(
        pl.kernel, mesh=mesh,
        out_type=[
            jax.ShapeDtypeStruct((npad, mw), jnp.int32),
            jax.ShapeDtypeStruct((npad, ew), jnp.float32),
        ],
        scratch_types=[
            pltpu.VMEM((chunk,), jnp.int32),
            pltpu.VMEM((chunk, mw), jnp.int32),
            pltpu.VMEM((chunk, ew), jnp.float32),
            pltpu.SemaphoreType.DMA,
        ],
    )
    def scatter_k(mask_hbm, ex_hbm, rank_hbm, omask_hbm, oex_hbm,
                  idx_v, mrows_v, erows_v, sem):
        wid = lax.axis_index("s") * info.num_cores + lax.axis_index("c")
        for c in range(nchunk):
            base = wid * rows_per_w + c * chunk
            pltpu.sync_copy(rank_hbm.at[pl.ds(base, chunk)], idx_v)
            pltpu.sync_copy(mask_hbm.at[pl.ds(base, chunk)], mrows_v)
            pltpu.sync_copy(ex_hbm.at[pl.ds(base, chunk)], erows_v)
            cp1 = pltpu.async_copy(mrows_v, omask_hbm.at[idx_v], sem)
            cp2 = pltpu.async_copy(erows_v, oex_hbm.at[idx_v], sem)
            cp1.wait()
            cp2.wait()

    return scatter_k(mask_words, extras, rank)


def _nms_cell(labA_ref, labB_ref, scoB_ref, a_ref, b_ref,
              keep_out, ks_out, keep_scr, sup_ref, s_ref, kl_ref):
    bi = pl.program_id(0)
    bj = pl.program_id(1)

    @pl.when(bj == 0)
    def _init():
        sup_ref[...] = jnp.zeros_like(sup_ref)

    @pl.when(bj <= bi)
    def _work():
        A = a_ref[...].astype(jnp.bfloat16)
        B = b_ref[...].astype(jnp.bfloat16)
        inter = jax.lax.dot_general(
            A, B, dimension_numbers=(((1,), (1,)), ((), ())),
            preferred_element_type=jnp.float32)
        areaA = jnp.sum(A.astype(jnp.float32), axis=1)
        areaB = jnp.sum(B.astype(jnp.float32), axis=1)
        labA = labA_ref[0, 0, :]
        labB = labB_ref[0, 0, :]
        cond = (3.0 * inter > areaA[:, None] + areaB[None, :]) \
            & (labA[:, None] == labB[None, :])
        condf = cond.astype(jnp.float32)

        @pl.when(bj < bi)
        def _offdiag():
            kb = keep_scr[pl.ds(bj, 1), :]
            contrib = jax.lax.dot_general(
                kb, condf, dimension_numbers=(((1,), (0,)), ((), ())),
                preferred_element_type=jnp.float32)
            sup_ref[...] = sup_ref[...] + contrib

        @pl.when(bj == bi)
        def _diag():
            ii = jax.lax.broadcasted_iota(jnp.int32, (BLK, BLK), 0)
            jj = jax.lax.broadcasted_iota(jnp.int32, (BLK, BLK), 1)
            Sf = condf * (jj > ii).astype(jnp.float32)
            s_ref[...] = Sf
            kl_ref[...] = (sup_ref[...] == 0.0).astype(jnp.float32)

            @pl.when(jnp.max(Sf) > 0.0)
            def _serial():
                lane = jax.lax.broadcasted_iota(jnp.int32, (1, BLK), 1)

                def body(i, carry):
                    kl = kl_ref[...]
                    ki = jnp.sum(jnp.where(lane == i, kl, 0.0))
                    srow = s_ref[pl.ds(i, 1), :]
                    kl_ref[...] = kl * (1.0 - srow * ki)
                    return carry

                jax.lax.fori_loop(0, BLK, body, 0)

            kl = kl_ref[...]
            keep_scr[pl.ds(bi, 1), :] = kl
            keep_out[0, ...] = kl
            ks_out[0, ...] = kl * scoB_ref[0, ...]


def _run_nms(msp, labp, scop, nb):
    grid = (nb, nb)
    out_shape = [
        jax.ShapeDtypeStruct((nb, 1, BLK), jnp.float32),
        jax.ShapeDtypeStruct((nb, 1, BLK), jnp.float32),
    ]
    keep_f, ks = pl.pallas_call(
        _nms_cell,
        grid=grid,
        in_specs=[
            pl.BlockSpec((1, 1, BLK), lambda i, j: (jnp.minimum(i, j), 0, 0)),
            pl.BlockSpec((1, 1, BLK), lambda i, j: (i, 0, 0)),
            pl.BlockSpec((1, 1, BLK), lambda i, j: (i, 0, 0)),
            pl.BlockSpec((BLK, MP), lambda i, j: (jnp.minimum(i, j), 0)),
            pl.BlockSpec((BLK, MP), lambda i, j: (i, 0)),
        ],
        out_specs=[
            pl.BlockSpec((1, 1, BLK), lambda i, j: (i, 0, 0)),
            pl.BlockSpec((1, 1, BLK), lambda i, j: (i, 0, 0)),
        ],
        out_shape=out_shape,
        scratch_shapes=[
            pltpu.VMEM((nb, BLK), jnp.float32),
            pltpu.VMEM((1, BLK), jnp.float32),
            pltpu.VMEM((BLK, BLK), jnp.float32),
            pltpu.VMEM((1, BLK), jnp.float32),
        ],
        compiler_params=pltpu.CompilerParams(
            dimension_semantics=("arbitrary", "arbitrary")),
    )(labp, labp, scop, msp, msp)
    return keep_f, ks


def kernel(bev_masks, scores, labels):
    n, m = bev_masks.shape
    nb = (n + BLK - 1) // BLK
    npad = nb * BLK

    scop = jnp.full((npad,), -1.0, jnp.float32).at[:n].set(scores)
    labp = jnp.full((npad,), -1.0, jnp.float32).at[:n].set(
        labels.astype(jnp.float32))

    mwords = _pack(bev_masks, n, m, npad, nb)

    rank, extras = _ranks(scop, labp, npad, nb)

    smw, sex = _sc_sort_scatter(mwords, extras, rank.reshape(npad), npad)

    msp = _unpack(smw, npad, nb)

    scos = sex[:, 0]
    labs = sex[:, 1].astype(jnp.int32)
    order = sex[:n, 2].astype(jnp.int32)

    keep_f, ks = _run_nms(msp, labs.reshape(nb, 1, BLK),
                          scos.reshape(nb, 1, BLK), nb)

    keep = keep_f.reshape(npad)[:n] > 0.5
    kept_scores = ks.reshape(npad)[:n]
    return order, keep, kept_scores


# speedup vs baseline: 2.9314x; 1.0296x over previous
import functools

import jax
import jax.numpy as jnp
from jax import lax
from jax.experimental import pallas as pl
from jax.experimental.pallas import tpu as pltpu
from jax.experimental.pallas import tpu_sc as plsc

BLK = 1024
MP = 2560
EW = 128


def _rank_cell(s_blk_ref, s_all_ref, lab_blk_ref, rank_ref, ex_ref):
    bi = pl.program_id(0)
    si = s_blk_ref[0, 0, :][:, None]
    sall = s_all_ref[...]
    gt = (sall > si).astype(jnp.float32)
    jglob = jax.lax.broadcasted_iota(jnp.int32, gt.shape, 1)
    iglob = bi * BLK + jax.lax.broadcasted_iota(jnp.int32, gt.shape, 0)
    eq = ((sall == si) & (jglob < iglob)).astype(jnp.float32)
    rank_ref[0, 0, :] = jnp.sum(gt + eq, axis=1).astype(jnp.int32)

    lane = jax.lax.broadcasted_iota(jnp.int32, (BLK, EW), 1)
    row_i = (bi * BLK
             + jax.lax.broadcasted_iota(jnp.int32, (BLK, EW), 0)
             ).astype(jnp.float32)
    labc = lab_blk_ref[0, 0, :][:, None]
    ex = jnp.where(lane == 0, si, 0.0)
    ex = jnp.where(lane == 1, labc, ex)
    ex_ref[...] = jnp.where(lane == 2, row_i, ex)


def _ranks(scop, labp, npad, nb):
    return pl.pallas_call(
        _rank_cell,
        grid=(nb,),
        in_specs=[
            pl.BlockSpec((1, 1, BLK), lambda i: (i, 0, 0)),
            pl.BlockSpec((1, npad), lambda i: (0, 0)),
            pl.BlockSpec((1, 1, BLK), lambda i: (i, 0, 0)),
        ],
        out_specs=[
            pl.BlockSpec((1, 1, BLK), lambda i: (i, 0, 0)),
            pl.BlockSpec((BLK, EW), lambda i: (i, 0)),
        ],
        out_shape=[
            jax.ShapeDtypeStruct((nb, 1, BLK), jnp.int32),
            jax.ShapeDtypeStruct((npad, EW), jnp.float32),
        ],
        compiler_params=pltpu.CompilerParams(
            dimension_semantics=("arbitrary",)),
    )(scop.reshape(nb, 1, BLK), scop.reshape(1, npad),
      labp.reshape(nb, 1, BLK))


def _pack(masks, n, m, npad, nb):
    qw = MP // 4

    def cell(m_ref, w_ref):
        bi = pl.program_id(0)
        x = m_ref[...].astype(jnp.int32)
        rglob = bi * BLK + jax.lax.broadcasted_iota(jnp.int32, (BLK, MP), 0)
        cglob = jax.lax.broadcasted_iota(jnp.int32, (BLK, MP), 1)
        x = jnp.where((rglob < n) & (cglob < m), x, 0)
        w_ref[...] = (x[:, 0:qw]
                      | (x[:, qw:2 * qw] << 8)
                      | (x[:, 2 * qw:3 * qw] << 16)
                      | (x[:, 3 * qw:4 * qw] << 24))

    return pl.pallas_call(
        cell,
        grid=(nb,),
        in_specs=[pl.BlockSpec((BLK, MP), lambda i: (i, 0))],
        out_specs=pl.BlockSpec((BLK, qw), lambda i: (i, 0)),
        out_shape=jax.ShapeDtypeStruct((npad, qw), jnp.int32),
        compiler_params=pltpu.CompilerParams(
            dimension_semantics=("arbitrary",)),
    )(masks)


def _unpack_cell(w_ref, o_ref):
    w = w_ref[...]
    planes = [((w >> (8 * k)) & 1).astype(jnp.int8) for k in range(4)]
    o_ref[...] = jnp.concatenate(planes, axis=1)


def _unpack(smw, npad, nb):
    return pl.pallas_call(
        _unpack_cell,
        grid=(nb,),
        in_specs=[pl.BlockSpec((BLK, MP // 4), lambda i: (i, 0))],
        out_specs=pl.BlockSpec((BLK, MP), lambda i: (i, 0)),
        out_shape=jax.ShapeDtypeStruct((npad, MP), jnp.int8),
        compiler_params=pltpu.CompilerParams(
            dimension_semantics=("arbitrary",)),
    )(smw)


def _sc_sort_scatter(mask_words, extras, rank, npad):
    info = plsc.get_sparse_core_info()
    nw = info.num_cores * info.num_subcores
    rows_per_w = npad // nw
    chunk = 80
    nchunk = rows_per_w // chunk
    mw = mask_words.shape[1]
    ew = extras.shape[1]
    mesh = plsc.VectorSubcoreMesh(core_axis_name="c", subcore_axis_name="s")

    @functools.partial(
        pl.kernel, mesh=mesh,
        out_type=[
            jax.ShapeDtypeStruct((npad, mw), jnp.int32),
            jax.ShapeDtypeStruct((npad, ew), jnp.float32),
        ],
        scratch_types=[
            pltpu.VMEM((chunk,), jnp.int32),
            pltpu.VMEM((chunk, mw), jnp.int32),
            pltpu.VMEM((chunk, ew), jnp.float32),
            pltpu.SemaphoreType.DMA,
        ],
    )
    def scatter_k(mask_hbm, ex_hbm, rank_hbm, omask_hbm, oex_hbm,
                  idx_v, mrows_v, erows_v, sem):
        wid = lax.axis_index("s") * info.num_cores + lax.axis_index("c")
        for c in range(nchunk):
            base = wid * rows_per_w + c * chunk
            pltpu.sync_copy(rank_hbm.at[pl.ds(base, chunk)], idx_v)
            pltpu.sync_copy(mask_hbm.at[pl.ds(base, chunk)], mrows_v)
            pltpu.sync_copy(ex_hbm.at[pl.ds(base, chunk)], erows_v)
            cp1 = pltpu.async_copy(mrows_v, omask_hbm.at[idx_v], sem)
            cp2 = pltpu.async_copy(erows_v, oex_hbm.at[idx_v], sem)
            cp1.wait()
            cp2.wait()

    return scatter_k(mask_words, extras, rank)


def _nms_cell(labA_ref, labB_ref, scoB_ref, a_ref, b_ref,
              keep_out, ks_out, keep_scr, sup_ref, s_ref, kl_ref):
    bi = pl.program_id(0)
    bj = pl.program_id(1)

    @pl.when(bj == 0)
    def _init():
        sup_ref[...] = jnp.zeros_like(sup_ref)

    @pl.when(bj <= bi)
    def _work():
        A = a_ref[...].astype(jnp.bfloat16)
        B = b_ref[...].astype(jnp.bfloat16)
        inter = jax.lax.dot_general(
            A, B, dimension_numbers=(((1,), (1,)), ((), ())),
            preferred_element_type=jnp.float32)
        areaA = jnp.sum(A.astype(jnp.float32), axis=1)
        areaB = jnp.sum(B.astype(jnp.float32), axis=1)
        labA = labA_ref[0, 0, :]
        labB = labB_ref[0, 0, :]
        cond = (3.0 * inter > areaA[:, None] + areaB[None, :]) \
            & (labA[:, None] == labB[None, :])
        condf = cond.astype(jnp.float32)

        @pl.when(bj < bi)
        def _offdiag():
            kb = keep_scr[pl.ds(bj, 1), :]
            contrib = jax.lax.dot_general(
                kb, condf, dimension_numbers=(((1,), (0,)), ((), ())),
                preferred_element_type=jnp.float32)
            sup_ref[...] = sup_ref[...] + contrib

        @pl.when(bj == bi)
        def _diag():
            ii = jax.lax.broadcasted_iota(jnp.int32, (BLK, BLK), 0)
            jj = jax.lax.broadcasted_iota(jnp.int32, (BLK, BLK), 1)
            Sf = condf * (jj > ii).astype(jnp.float32)
            s_ref[...] = Sf
            kl_ref[...] = (sup_ref[...] == 0.0).astype(jnp.float32)

            @pl.when(jnp.max(Sf) > 0.0)
            def _serial():
                lane = jax.lax.broadcasted_iota(jnp.int32, (1, BLK), 1)

                def body(i, carry):
                    kl = kl_ref[...]
                    ki = jnp.sum(jnp.where(lane == i, kl, 0.0))
                    srow = s_ref[pl.ds(i, 1), :]
                    kl_ref[...] = kl * (1.0 - srow * ki)
                    return carry

                jax.lax.fori_loop(0, BLK, body, 0)

            kl = kl_ref[...]
            keep_scr[pl.ds(bi, 1), :] = kl
            keep_out[0, ...] = kl
            ks_out[0, ...] = kl * scoB_ref[0, ...]


def _run_nms(msp, labp, scop, nb):
    grid = (nb, nb)
    out_shape = [
        jax.ShapeDtypeStruct((nb, 1, BLK), jnp.float32),
        jax.ShapeDtypeStruct((nb, 1, BLK), jnp.float32),
    ]
    keep_f, ks = pl.pallas_call(
        _nms_cell,
        grid=grid,
        in_specs=[
            pl.BlockSpec((1, 1, BLK), lambda i, j: (jnp.minimum(i, j), 0, 0)),
            pl.BlockSpec((1, 1, BLK), lambda i, j: (i, 0, 0)),
            pl.BlockSpec((1, 1, BLK), lambda i, j: (i, 0, 0)),
            pl.BlockSpec((BLK, MP), lambda i, j: (jnp.minimum(i, j), 0)),
            pl.BlockSpec((BLK, MP), lambda i, j: (i, 0)),
        ],
        out_specs=[
            pl.BlockSpec((1, 1, BLK), lambda i, j: (i, 0, 0)),
            pl.BlockSpec((1, 1, BLK), lambda i, j: (i, 0, 0)),
        ],
        out_shape=out_shape,
        scratch_shapes=[
            pltpu.VMEM((nb, BLK), jnp.float32),
            pltpu.VMEM((1, BLK), jnp.float32),
            pltpu.VMEM((BLK, BLK), jnp.float32),
            pltpu.VMEM((1, BLK), jnp.float32),
        ],
        compiler_params=pltpu.CompilerParams(
            dimension_semantics=("arbitrary", "arbitrary")),
    )(labp, labp, scop, msp, msp)
    return keep_f, ks


def kernel(bev_masks, scores, labels):
    n, m = bev_masks.shape
    nb = (n + BLK - 1) // BLK
    npad = nb * BLK

    scop = jnp.full((npad,), -1.0, jnp.float32).at[:n].set(scores)
    labp = jnp.full((npad,), -1.0, jnp.float32).at[:n].set(
        labels.astype(jnp.float32))

    mwords = _pack(bev_masks, n, m, npad, nb)

    rank, extras = _ranks(scop, labp, npad, nb)

    smw, sex = _sc_sort_scatter(mwords, extras, rank.reshape(npad), npad)

    msp = _unpack(smw, npad, nb)

    scos = sex[:, 0]
    labs = sex[:, 1].astype(jnp.int32)
    order = sex[:n, 2].astype(jnp.int32)

    keep_f, ks = _run_nms(msp, labs.reshape(nb, 1, BLK),
                          scos.reshape(nb, 1, BLK), nb)

    keep = keep_f.reshape(npad)[:n] > 0.5
    kept_scores = ks.reshape(npad)[:n]
    return order, keep, kept_scores


# speedup vs baseline: 2.9432x; 1.0040x over previous
import functools

import jax
import jax.numpy as jnp
from jax import lax
from jax.experimental import pallas as pl
from jax.experimental.pallas import tpu as pltpu
from jax.experimental.pallas import tpu_sc as plsc

BLK = 1024
MP = 2560
EW = 128


def _rank_cell(s_blk_ref, s_all_ref, lab_blk_ref, rank_ref, ex_ref):
    bi = pl.program_id(0)
    si = s_blk_ref[0, 0, :][:, None]
    sall = s_all_ref[...]
    gt = (sall > si).astype(jnp.float32)
    jglob = jax.lax.broadcasted_iota(jnp.int32, gt.shape, 1)
    iglob = bi * BLK + jax.lax.broadcasted_iota(jnp.int32, gt.shape, 0)
    eq = ((sall == si) & (jglob < iglob)).astype(jnp.float32)
    rank_ref[0, 0, :] = jnp.sum(gt + eq, axis=1).astype(jnp.int32)

    lane = jax.lax.broadcasted_iota(jnp.int32, (BLK, EW), 1)
    row_i = (bi * BLK
             + jax.lax.broadcasted_iota(jnp.int32, (BLK, EW), 0)
             ).astype(jnp.float32)
    labc = lab_blk_ref[0, 0, :][:, None]
    ex = jnp.where(lane == 0, si, 0.0)
    ex = jnp.where(lane == 1, labc, ex)
    ex_ref[...] = jnp.where(lane == 2, row_i, ex)


def _ranks(scop, labp, npad, nb):
    return pl.pallas_call(
        _rank_cell,
        grid=(nb,),
        in_specs=[
            pl.BlockSpec((1, 1, BLK), lambda i: (i, 0, 0)),
            pl.BlockSpec((1, npad), lambda i: (0, 0)),
            pl.BlockSpec((1, 1, BLK), lambda i: (i, 0, 0)),
        ],
        out_specs=[
            pl.BlockSpec((1, 1, BLK), lambda i: (i, 0, 0)),
            pl.BlockSpec((BLK, EW), lambda i: (i, 0)),
        ],
        out_shape=[
            jax.ShapeDtypeStruct((nb, 1, BLK), jnp.int32),
            jax.ShapeDtypeStruct((npad, EW), jnp.float32),
        ],
        compiler_params=pltpu.CompilerParams(
            dimension_semantics=("arbitrary",)),
    )(scop.reshape(nb, 1, BLK), scop.reshape(1, npad),
      labp.reshape(nb, 1, BLK))


def _pack(masks, n, m, npad, nb):
    qw = MP // 4

    def cell(m_ref, w_ref):
        bi = pl.program_id(0)
        x = m_ref[...].astype(jnp.int32)
        rglob = bi * BLK + jax.lax.broadcasted_iota(jnp.int32, (BLK, MP), 0)
        cglob = jax.lax.broadcasted_iota(jnp.int32, (BLK, MP), 1)
        x = jnp.where((rglob < n) & (cglob < m), x, 0)
        w_ref[...] = (x[:, 0:qw]
                      | (x[:, qw:2 * qw] << 8)
                      | (x[:, 2 * qw:3 * qw] << 16)
                      | (x[:, 3 * qw:4 * qw] << 24))

    return pl.pallas_call(
        cell,
        grid=(nb,),
        in_specs=[pl.BlockSpec((BLK, MP), lambda i: (i, 0))],
        out_specs=pl.BlockSpec((BLK, qw), lambda i: (i, 0)),
        out_shape=jax.ShapeDtypeStruct((npad, qw), jnp.int32),
        compiler_params=pltpu.CompilerParams(
            dimension_semantics=("arbitrary",)),
    )(masks)


def _unpack_cell(w_ref, o_ref):
    w = w_ref[...]
    planes = [((w >> (8 * k)) & 1).astype(jnp.int8) for k in range(4)]
    o_ref[...] = jnp.concatenate(planes, axis=1)


def _unpack(smw, npad, nb):
    return pl.pallas_call(
        _unpack_cell,
        grid=(nb,),
        in_specs=[pl.BlockSpec((BLK, MP // 4), lambda i: (i, 0))],
        out_specs=pl.BlockSpec((BLK, MP), lambda i: (i, 0)),
        out_shape=jax.ShapeDtypeStruct((npad, MP), jnp.int8),
        compiler_params=pltpu.CompilerParams(
            dimension_semantics=("arbitrary",)),
    )(smw)


def _sc_sort_scatter(mask_words, extras, rank, npad):
    info = plsc.get_sparse_core_info()
    nw = info.num_cores * info.num_subcores
    rows_per_w = npad // nw
    chunk = 80
    nchunk = rows_per_w // chunk
    mw = mask_words.shape[1]
    ew = extras.shape[1]
    mesh = plsc.VectorSubcoreMesh(core_axis_name="c", subcore_axis_name="s")

    @functools.partial(
        pl.kernel, mesh=mesh,
        out_type=[
            jax.ShapeDtypeStruct((npad, mw), jnp.int32),
            jax.ShapeDtypeStruct((npad, ew), jnp.float32),
        ],
        scratch_types=[
            pltpu.VMEM((chunk,), jnp.int32),
            pltpu.VMEM((chunk, mw), jnp.int32),
            pltpu.VMEM((chunk, ew), jnp.float32),
            pltpu.SemaphoreType.DMA,
        ],
    )
    def scatter_k(mask_hbm, ex_hbm, rank_hbm, omask_hbm, oex_hbm,
                  idx_v, mrows_v, erows_v, sem):
        wid = lax.axis_index("s") * info.num_cores + lax.axis_index("c")
        for c in range(nchunk):
            base = wid * rows_per_w + c * chunk
            pltpu.sync_copy(rank_hbm.at[pl.ds(base, chunk)], idx_v)
            pltpu.sync_copy(mask_hbm.at[pl.ds(base, chunk)], mrows_v)
            pltpu.sync_copy(ex_hbm.at[pl.ds(base, chunk)], erows_v)
            cp1 = pltpu.async_copy(mrows_v, omask_hbm.at[idx_v], sem)
            cp2 = pltpu.async_copy(erows_v, oex_hbm.at[idx_v], sem)
            cp1.wait()
            cp2.wait()

    return scatter_k(mask_words, extras, rank)


def _nms_cell(labA_ref, labB_ref, scoB_ref, a_ref, b_ref,
              keep_out, ks_out, keep_scr, sup_ref, s_ref, kl_ref):
    bi = pl.program_id(0)
    bj = pl.program_id(1)

    @pl.when(bj == 0)
    def _init():
        sup_ref[...] = jnp.zeros_like(sup_ref)

    @pl.when(bj <= bi)
    def _work():
        A = a_ref[...]
        B = b_ref[...]
        inter = jax.lax.dot_general(
            A, B, dimension_numbers=(((1,), (1,)), ((), ())),
            preferred_element_type=jnp.int32)
        areaA = jnp.sum(A.astype(jnp.int32), axis=1)
        areaB = jnp.sum(B.astype(jnp.int32), axis=1)
        labA = labA_ref[0, 0, :]
        labB = labB_ref[0, 0, :]
        cond = (3 * inter > areaA[:, None] + areaB[None, :]) \
            & (labA[:, None] == labB[None, :])
        condf = cond.astype(jnp.float32)

        @pl.when(bj < bi)
        def _offdiag():
            kb = keep_scr[pl.ds(bj, 1), :]
            contrib = jax.lax.dot_general(
                kb, condf, dimension_numbers=(((1,), (0,)), ((), ())),
                preferred_element_type=jnp.float32)
            sup_ref[...] = sup_ref[...] + contrib

        @pl.when(bj == bi)
        def _diag():
            ii = jax.lax.broadcasted_iota(jnp.int32, (BLK, BLK), 0)
            jj = jax.lax.broadcasted_iota(jnp.int32, (BLK, BLK), 1)
            Sf = condf * (jj > ii).astype(jnp.float32)
            s_ref[...] = Sf
            kl_ref[...] = (sup_ref[...] == 0.0).astype(jnp.float32)

            @pl.when(jnp.max(Sf) > 0.0)
            def _serial():
                lane = jax.lax.broadcasted_iota(jnp.int32, (1, BLK), 1)

                def body(i, carry):
                    kl = kl_ref[...]
                    ki = jnp.sum(jnp.where(lane == i, kl, 0.0))
                    srow = s_ref[pl.ds(i, 1), :]
                    kl_ref[...] = kl * (1.0 - srow * ki)
                    return carry

                jax.lax.fori_loop(0, BLK, body, 0)

            kl = kl_ref[...]
            keep_scr[pl.ds(bi, 1), :] = kl
            keep_out[0, ...] = kl
            ks_out[0, ...] = kl * scoB_ref[0, ...]


def _run_nms(msp, labp, scop, nb):
    grid = (nb, nb)
    out_shape = [
        jax.ShapeDtypeStruct((nb, 1, BLK), jnp.float32),
        jax.ShapeDtypeStruct((nb, 1, BLK), jnp.float32),
    ]
    keep_f, ks = pl.pallas_call(
        _nms_cell,
        grid=grid,
        in_specs=[
            pl.BlockSpec((1, 1, BLK), lambda i, j: (jnp.minimum(i, j), 0, 0)),
            pl.BlockSpec((1, 1, BLK), lambda i, j: (i, 0, 0)),
            pl.BlockSpec((1, 1, BLK), lambda i, j: (i, 0, 0)),
            pl.BlockSpec((BLK, MP), lambda i, j: (jnp.minimum(i, j), 0)),
            pl.BlockSpec((BLK, MP), lambda i, j: (i, 0)),
        ],
        out_specs=[
            pl.BlockSpec((1, 1, BLK), lambda i, j: (i, 0, 0)),
            pl.BlockSpec((1, 1, BLK), lambda i, j: (i, 0, 0)),
        ],
        out_shape=out_shape,
        scratch_shapes=[
            pltpu.VMEM((nb, BLK), jnp.float32),
            pltpu.VMEM((1, BLK), jnp.float32),
            pltpu.VMEM((BLK, BLK), jnp.float32),
            pltpu.VMEM((1, BLK), jnp.float32),
        ],
        compiler_params=pltpu.CompilerParams(
            dimension_semantics=("arbitrary", "arbitrary")),
    )(labp, labp, scop, msp, msp)
    return keep_f, ks


def kernel(bev_masks, scores, labels):
    n, m = bev_masks.shape
    nb = (n + BLK - 1) // BLK
    npad = nb * BLK

    scop = jnp.full((npad,), -1.0, jnp.float32).at[:n].set(scores)
    labp = jnp.full((npad,), -1.0, jnp.float32).at[:n].set(
        labels.astype(jnp.float32))

    mwords = _pack(bev_masks, n, m, npad, nb)

    rank, extras = _ranks(scop, labp, npad, nb)

    smw, sex = _sc_sort_scatter(mwords, extras, rank.reshape(npad), npad)

    msp = _unpack(smw, npad, nb)

    scos = sex[:, 0]
    labs = sex[:, 1].astype(jnp.int32)
    order = sex[:n, 2].astype(jnp.int32)

    keep_f, ks = _run_nms(msp, labs.reshape(nb, 1, BLK),
                          scos.reshape(nb, 1, BLK), nb)

    keep = keep_f.reshape(npad)[:n] > 0.5
    kept_scores = ks.reshape(npad)[:n]
    return order, keep, kept_scores


# speedup vs baseline: 3.0188x; 1.0257x over previous
import functools

import jax
import jax.numpy as jnp
from jax import lax
from jax.experimental import pallas as pl
from jax.experimental.pallas import tpu as pltpu
from jax.experimental.pallas import tpu_sc as plsc

BLK = 1024
MP = 2560
EW = 128


def _rank_cell(s_blk_ref, s_all_ref, lab_blk_ref, rank_ref, ex_ref):
    bi = pl.program_id(0)
    si = s_blk_ref[0, 0, :][:, None]
    sall = s_all_ref[...]
    gt = (sall > si).astype(jnp.float32)
    jglob = jax.lax.broadcasted_iota(jnp.int32, gt.shape, 1)
    iglob = bi * BLK + jax.lax.broadcasted_iota(jnp.int32, gt.shape, 0)
    eq = ((sall == si) & (jglob < iglob)).astype(jnp.float32)
    rank_ref[0, 0, :] = jnp.sum(gt + eq, axis=1).astype(jnp.int32)

    lane = jax.lax.broadcasted_iota(jnp.int32, (BLK, EW), 1)
    row_i = (bi * BLK
             + jax.lax.broadcasted_iota(jnp.int32, (BLK, EW), 0)
             ).astype(jnp.float32)
    labc = lab_blk_ref[0, 0, :][:, None]
    ex = jnp.where(lane == 0, si, 0.0)
    ex = jnp.where(lane == 1, labc, ex)
    ex_ref[...] = jnp.where(lane == 2, row_i, ex)


def _ranks(scop, labp, npad, nb):
    return pl.pallas_call(
        _rank_cell,
        grid=(nb,),
        in_specs=[
            pl.BlockSpec((1, 1, BLK), lambda i: (i, 0, 0)),
            pl.BlockSpec((1, npad), lambda i: (0, 0)),
            pl.BlockSpec((1, 1, BLK), lambda i: (i, 0, 0)),
        ],
        out_specs=[
            pl.BlockSpec((1, 1, BLK), lambda i: (i, 0, 0)),
            pl.BlockSpec((BLK, EW), lambda i: (i, 0)),
        ],
        out_shape=[
            jax.ShapeDtypeStruct((nb, 1, BLK), jnp.int32),
            jax.ShapeDtypeStruct((npad, EW), jnp.float32),
        ],
        compiler_params=pltpu.CompilerParams(
            dimension_semantics=("arbitrary",)),
    )(scop.reshape(nb, 1, BLK), scop.reshape(1, npad),
      labp.reshape(nb, 1, BLK))


def _pack(masks, n, m, npad, nb):
    qw = MP // 4

    def cell(m_ref, w_ref):
        bi = pl.program_id(0)
        x = m_ref[...].astype(jnp.int32)
        rglob = bi * BLK + jax.lax.broadcasted_iota(jnp.int32, (BLK, MP), 0)
        cglob = jax.lax.broadcasted_iota(jnp.int32, (BLK, MP), 1)
        x = jnp.where((rglob < n) & (cglob < m), x, 0)
        w_ref[...] = (x[:, 0:qw]
                      | (x[:, qw:2 * qw] << 8)
                      | (x[:, 2 * qw:3 * qw] << 16)
                      | (x[:, 3 * qw:4 * qw] << 24))

    return pl.pallas_call(
        cell,
        grid=(nb,),
        in_specs=[pl.BlockSpec((BLK, MP), lambda i: (i, 0))],
        out_specs=pl.BlockSpec((BLK, qw), lambda i: (i, 0)),
        out_shape=jax.ShapeDtypeStruct((npad, qw), jnp.int32),
        compiler_params=pltpu.CompilerParams(
            dimension_semantics=("arbitrary",)),
    )(masks)


def _unpack_cell(w_ref, o_ref, arow_ref, acol_ref):
    w = w_ref[...]
    planes = [((w >> (8 * k)) & 1).astype(jnp.int8) for k in range(4)]
    o_ref[...] = jnp.concatenate(planes, axis=1)
    area = sum(jnp.sum((w >> (8 * k)) & 1, axis=1) for k in range(4))
    arow_ref[0, 0, :] = area
    acol_ref[...] = jnp.broadcast_to(area[:, None], (BLK, 8))


def _unpack(smw, npad, nb):
    return pl.pallas_call(
        _unpack_cell,
        grid=(nb,),
        in_specs=[pl.BlockSpec((BLK, MP // 4), lambda i: (i, 0))],
        out_specs=[
            pl.BlockSpec((BLK, MP), lambda i: (i, 0)),
            pl.BlockSpec((1, 1, BLK), lambda i: (i, 0, 0)),
            pl.BlockSpec((BLK, 8), lambda i: (i, 0)),
        ],
        out_shape=[
            jax.ShapeDtypeStruct((npad, MP), jnp.int8),
            jax.ShapeDtypeStruct((nb, 1, BLK), jnp.int32),
            jax.ShapeDtypeStruct((npad, 8), jnp.int32),
        ],
        compiler_params=pltpu.CompilerParams(
            dimension_semantics=("arbitrary",)),
    )(smw)


def _sc_sort_scatter(mask_words, extras, rank, npad):
    info = plsc.get_sparse_core_info()
    nw = info.num_cores * info.num_subcores
    rows_per_w = npad // nw
    chunk = 80
    nchunk = rows_per_w // chunk
    mw = mask_words.shape[1]
    ew = extras.shape[1]
    mesh = plsc.VectorSubcoreMesh(core_axis_name="c", subcore_axis_name="s")

    @functools.partial(
        pl.kernel, mesh=mesh,
        out_type=[
            jax.ShapeDtypeStruct((npad, mw), jnp.int32),
            jax.ShapeDtypeStruct((npad, ew), jnp.float32),
        ],
        scratch_types=[
            pltpu.VMEM((chunk,), jnp.int32),
            pltpu.VMEM((chunk, mw), jnp.int32),
            pltpu.VMEM((chunk, ew), jnp.float32),
            pltpu.SemaphoreType.DMA,
        ],
    )
    def scatter_k(mask_hbm, ex_hbm, rank_hbm, omask_hbm, oex_hbm,
                  idx_v, mrows_v, erows_v, sem):
        wid = lax.axis_index("s") * info.num_cores + lax.axis_index("c")
        for c in range(nchunk):
            base = wid * rows_per_w + c * chunk
            pltpu.sync_copy(rank_hbm.at[pl.ds(base, chunk)], idx_v)
            pltpu.sync_copy(mask_hbm.at[pl.ds(base, chunk)], mrows_v)
            pltpu.sync_copy(ex_hbm.at[pl.ds(base, chunk)], erows_v)
            cp1 = pltpu.async_copy(mrows_v, omask_hbm.at[idx_v], sem)
            cp2 = pltpu.async_copy(erows_v, oex_hbm.at[idx_v], sem)
            cp1.wait()
            cp2.wait()

    return scatter_k(mask_words, extras, rank)


def _nms_cell(labA_ref, labB_ref, scoB_ref, aA_ref, aB_ref, a_ref, b_ref,
              keep_out, ks_out, keep_scr, sup_ref, s_ref, kl_ref):
    bi = pl.program_id(0)
    bj = pl.program_id(1)

    @pl.when(bj == 0)
    def _init():
        sup_ref[...] = jnp.zeros_like(sup_ref)

    @pl.when(bj <= bi)
    def _work():
        A = a_ref[...]
        B = b_ref[...]
        inter = jax.lax.dot_general(
            A, B, dimension_numbers=(((1,), (1,)), ((), ())),
            preferred_element_type=jnp.int32)
        areaA = aA_ref[...][:, 0:1]
        areaB = aB_ref[0, ...]
        labA = labA_ref[0, 0, :]
        labB = labB_ref[0, 0, :]
        cond = (3 * inter > areaA + areaB) \
            & (labA[:, None] == labB[None, :])
        condf = cond.astype(jnp.bfloat16)

        @pl.when(bj < bi)
        def _offdiag():
            kb = keep_scr[pl.ds(bj, 1), :].astype(jnp.bfloat16)
            contrib = jax.lax.dot_general(
                kb, condf, dimension_numbers=(((1,), (0,)), ((), ())),
                preferred_element_type=jnp.float32)
            sup_ref[...] = sup_ref[...] + contrib

        @pl.when(bj == bi)
        def _diag():
            ii = jax.lax.broadcasted_iota(jnp.int32, (BLK, BLK), 0)
            jj = jax.lax.broadcasted_iota(jnp.int32, (BLK, BLK), 1)
            Sf = condf.astype(jnp.float32) * (jj > ii).astype(jnp.float32)
            s_ref[...] = Sf
            kl_ref[...] = (sup_ref[...] == 0.0).astype(jnp.float32)

            @pl.when(jnp.max(Sf) > 0.0)
            def _serial():
                lane = jax.lax.broadcasted_iota(jnp.int32, (1, BLK), 1)

                def body(i, carry):
                    kl = kl_ref[...]
                    ki = jnp.sum(jnp.where(lane == i, kl, 0.0))
                    srow = s_ref[pl.ds(i, 1), :]
                    kl_ref[...] = kl * (1.0 - srow * ki)
                    return carry

                jax.lax.fori_loop(0, BLK, body, 0)

            kl = kl_ref[...]
            keep_scr[pl.ds(bi, 1), :] = kl
            keep_out[0, ...] = kl
            ks_out[0, ...] = kl * scoB_ref[0, ...]


def _run_nms(msp, arow, acol, labp, scop, nb):
    grid = (nb, nb)
    out_shape = [
        jax.ShapeDtypeStruct((nb, 1, BLK), jnp.float32),
        jax.ShapeDtypeStruct((nb, 1, BLK), jnp.float32),
    ]
    keep_f, ks = pl.pallas_call(
        _nms_cell,
        grid=grid,
        in_specs=[
            pl.BlockSpec((1, 1, BLK), lambda i, j: (jnp.minimum(i, j), 0, 0)),
            pl.BlockSpec((1, 1, BLK), lambda i, j: (i, 0, 0)),
            pl.BlockSpec((1, 1, BLK), lambda i, j: (i, 0, 0)),
            pl.BlockSpec((BLK, 8), lambda i, j: (jnp.minimum(i, j), 0)),
            pl.BlockSpec((1, 1, BLK), lambda i, j: (i, 0, 0)),
            pl.BlockSpec((BLK, MP), lambda i, j: (jnp.minimum(i, j), 0)),
            pl.BlockSpec((BLK, MP), lambda i, j: (i, 0)),
        ],
        out_specs=[
            pl.BlockSpec((1, 1, BLK), lambda i, j: (i, 0, 0)),
            pl.BlockSpec((1, 1, BLK), lambda i, j: (i, 0, 0)),
        ],
        out_shape=out_shape,
        scratch_shapes=[
            pltpu.VMEM((nb, BLK), jnp.float32),
            pltpu.VMEM((1, BLK), jnp.float32),
            pltpu.VMEM((BLK, BLK), jnp.float32),
            pltpu.VMEM((1, BLK), jnp.float32),
        ],
        compiler_params=pltpu.CompilerParams(
            dimension_semantics=("arbitrary", "arbitrary")),
    )(labp, labp, scop, acol, arow, msp, msp)
    return keep_f, ks


def kernel(bev_masks, scores, labels):
    n, m = bev_masks.shape
    nb = (n + BLK - 1) // BLK
    npad = nb * BLK

    scop = jnp.full((npad,), -1.0, jnp.float32).at[:n].set(scores)
    labp = jnp.full((npad,), -1.0, jnp.float32).at[:n].set(
        labels.astype(jnp.float32))

    mwords = _pack(bev_masks, n, m, npad, nb)

    rank, extras = _ranks(scop, labp, npad, nb)

    smw, sex = _sc_sort_scatter(mwords, extras, rank.reshape(npad), npad)

    msp, arow, acol = _unpack(smw, npad, nb)

    scos = sex[:, 0]
    labs = sex[:, 1].astype(jnp.int32)
    order = sex[:n, 2].astype(jnp.int32)

    keep_f, ks = _run_nms(msp, arow, acol, labs.reshape(nb, 1, BLK),
                          scos.reshape(nb, 1, BLK), nb)

    keep = keep_f.reshape(npad)[:n] > 0.5
    kept_scores = ks.reshape(npad)[:n]
    return order, keep, kept_scores
